# baseline (speedup 1.0000x reference)
import functools

import jax
import jax.numpy as jnp
from jax import lax
from jax.experimental import pallas as pl
from jax.experimental.pallas import tpu as pltpu
from jax.experimental.pallas import tpu_sc as plsc

_N = 10000
_D = 128
_H = 64
_C = 16
_E = 320000

_NCORE = 2
_NSUB = 16
_SUB = 128
_NBJ = 80
_EPAD = _NCORE * _NSUB * _NBJ * _SUB
_NPAD = 10240
_RPS = _NPAD // _NSUB
_ZR = 128

_BMP = 2048
_BM = 2000

_SC_PARAMS = pltpu.CompilerParams(use_tc_tiling_on_sc=False)


def _make_agg(feat, with_deg, dt=jnp.float32, percore=False):
  part_t = jax.ShapeDtypeStruct((_NCORE, _NPAD, feat), dt)
  lanes = 16 if dt == jnp.float32 else 32
  out_type = ([part_t, jax.ShapeDtypeStruct((_NCORE, 1, _NPAD), jnp.float32)]
              if with_deg else part_t)
  scratch = [
      pltpu.VMEM((_NBJ, _SUB), jnp.int32),
      pltpu.VMEM((_NBJ, _SUB), jnp.int32),
      pltpu.VMEM((_SUB, feat), dt),
      pltpu.VMEM((_SUB, feat), dt),
      pltpu.VMEM((_ZR, feat), dt),
      pltpu.SemaphoreType.DMA,
      pltpu.SemaphoreType.DMA,
      pltpu.VMEM_SHARED((_NPAD, feat), dt),
  ]
  if with_deg:
    scratch.append(pltpu.VMEM_SHARED((_NPAD,), jnp.float32))
    scratch.append(pltpu.VMEM((_SUB,), jnp.float32))
    scratch.append(pltpu.SemaphoreType.DMA)

  def body(y_hbm, src_hbm, dst_hbm, *rest):
    if with_deg:
      (part_out, deg_out, src_v, dst_v, rows0, rows1, zrow_v, sem0, sem1,
       acc_sh, deg_sh, ones_v, dsem) = rest
    else:
      (part_out, src_v, dst_v, rows0, rows1, zrow_v, sem0, sem1,
       acc_sh) = rest
    c = lax.axis_index("c")
    s = lax.axis_index("s")
    sl = pl.ds(s * _RPS, _RPS)

    def zfill(r, carry):
      for k in range(feat // lanes):
        zrow_v[r, pl.ds(k * lanes, lanes)] = jnp.zeros((lanes,), dt)
      return carry

    lax.fori_loop(0, _ZR, zfill, 0)
    for m in range(_RPS // _ZR):
      pltpu.sync_copy(zrow_v, acc_sh.at[pl.ds(s * _RPS + m * _ZR, _ZR)])
    if with_deg:
      for i in range(_SUB // 16):
        ones_v[pl.ds(i * 16, 16)] = jnp.zeros((16,), jnp.float32)
      for m in range(_RPS // _SUB):
        pltpu.sync_copy(ones_v, deg_sh.at[pl.ds(s * _RPS + m * _SUB, _SUB)])
      for i in range(_SUB // 16):
        ones_v[pl.ds(i * 16, 16)] = jnp.ones((16,), jnp.float32)
    pltpu.sync_copy(src_hbm.at[c, s], src_v)
    pltpu.sync_copy(dst_hbm.at[c, s], dst_v)
    plsc.subcore_barrier()

    ytab = y_hbm.at[c] if percore else y_hbm
    pltpu.async_copy(ytab.at[src_v.at[0]], rows0, sem0)
    pltpu.async_copy(ytab.at[src_v.at[1]], rows1, sem1)

    def outer(jj, carry):
      for b, (rows, sem) in enumerate(((rows0, sem0), (rows1, sem1))):
        j = jj * 2 + b
        pltpu.make_async_copy(ytab.at[src_v.at[j]], rows, sem).wait()
        pltpu.sync_copy(rows, acc_sh.at[dst_v.at[j]], add=True)
        if with_deg:
          pltpu.async_copy(ones_v, deg_sh.at[dst_v.at[j]], dsem, add=True)

        @pl.when(j + 2 < _NBJ)
        def _():
          pltpu.async_copy(ytab.at[src_v.at[j + 2]], rows, sem)
      return carry

    lax.fori_loop(0, _NBJ // 2, outer, 0)
    if with_deg:
      def drain(j, carry):
        pltpu.make_async_copy(ones_v, deg_sh.at[dst_v.at[j]], dsem).wait()
        return carry
      lax.fori_loop(0, _NBJ, drain, 0)
    plsc.subcore_barrier()
    pltpu.sync_copy(acc_sh.at[sl], part_out.at[c, sl])
    if with_deg:
      pltpu.sync_copy(deg_sh.at[sl], deg_out.at[c, 0, sl])

  return functools.partial(
      pl.kernel, out_type=out_type,
      mesh=plsc.VectorSubcoreMesh(core_axis_name="c", subcore_axis_name="s"),
      scratch_types=scratch, compiler_params=_SC_PARAMS)(body)


_agg_h = _make_agg(_H, True, jnp.bfloat16, percore=True)
_agg_c = _make_agg(_C, False)


def _mm1_body(x_ref, w_ref, o_ref, ob_ref):
  y = jnp.dot(x_ref[...], w_ref[...], preferred_element_type=jnp.float32)
  o_ref[...] = y
  ob_ref[...] = jnp.broadcast_to(y.astype(jnp.bfloat16), (_NCORE, _BMP, _H))


_mm1 = pl.pallas_call(
    _mm1_body,
    grid=(_NPAD // _BMP,),
    in_specs=[pl.BlockSpec((_BMP, _D), lambda i: (i, 0)),
              pl.BlockSpec((_D, _H), lambda i: (0, 0))],
    out_specs=[pl.BlockSpec((_BMP, _H), lambda i: (i, 0)),
               pl.BlockSpec((_NCORE, _BMP, _H), lambda i: (0, i, 0))],
    out_shape=[jax.ShapeDtypeStruct((_NPAD, _H), jnp.float32),
               jax.ShapeDtypeStruct((_NCORE, _NPAD, _H), jnp.bfloat16)],
)


def _mid_body(y_ref, p0_ref, p1_ref, dg0_ref, dg1_ref, b1_ref, w2_ref,
              t_ref, dinv_ref):
  deg = jnp.transpose(dg0_ref[0] + dg1_ref[0])
  dinv = jnp.where(deg > 0.0, 1.0 / jnp.maximum(deg, 1.0), 0.0)
  aggv = (p0_ref[0].astype(jnp.float32)
          + p1_ref[0].astype(jnp.float32)) * dinv
  h = jnp.maximum(0.5 * y_ref[...] + 0.5 * aggv + b1_ref[...], 0.0)
  t_ref[...] = jnp.dot(h, w2_ref[...], preferred_element_type=jnp.float32)
  dinv_ref[...] = jnp.broadcast_to(dinv, (_BMP, _C))


_mid = pl.pallas_call(
    _mid_body,
    grid=(_NPAD // _BMP,),
    in_specs=[pl.BlockSpec((_BMP, _H), lambda i: (i, 0)),
              pl.BlockSpec((1, _BMP, _H), lambda i: (0, i, 0)),
              pl.BlockSpec((1, _BMP, _H), lambda i: (1, i, 0)),
              pl.BlockSpec((1, 1, _BMP), lambda i: (0, 0, i)),
              pl.BlockSpec((1, 1, _BMP), lambda i: (1, 0, i)),
              pl.BlockSpec((1, _H), lambda i: (0, 0)),
              pl.BlockSpec((_H, _C), lambda i: (0, 0))],
    out_specs=[pl.BlockSpec((_BMP, _C), lambda i: (i, 0)),
               pl.BlockSpec((_BMP, _C), lambda i: (i, 0))],
    out_shape=[jax.ShapeDtypeStruct((_NPAD, _C), jnp.float32),
               jax.ShapeDtypeStruct((_NPAD, _C), jnp.float32)],
)


def _fin_body(t_ref, q0_ref, q1_ref, dinv_ref, b2_ref, o_ref):
  aggv = (q0_ref[0] + q1_ref[0]) * dinv_ref[...]
  o_ref[...] = 0.5 * t_ref[...] + 0.5 * aggv + b2_ref[...]


_fin = pl.pallas_call(
    _fin_body,
    grid=(_N // _BM,),
    in_specs=[pl.BlockSpec((_BM, _C), lambda i: (i, 0)),
              pl.BlockSpec((1, _BM, _C), lambda i: (0, i, 0)),
              pl.BlockSpec((1, _BM, _C), lambda i: (1, i, 0)),
              pl.BlockSpec((_BM, _C), lambda i: (i, 0)),
              pl.BlockSpec((1, _C), lambda i: (0, 0))],
    out_specs=pl.BlockSpec((_BM, _C), lambda i: (i, 0)),
    out_shape=jax.ShapeDtypeStruct((_N, _C), jnp.float32),
)


def _pad_edges(idx, fill):
  return jnp.concatenate(
      [idx, jnp.full((_EPAD - _E,), fill, jnp.int32)]).reshape(
          _NCORE, _NSUB, _NBJ, _SUB)


def kernel(x, edge_index, W1, b1, W2, b2):
  src = edge_index[0]
  dst = edge_index[1]
  src_p = _pad_edges(src, 0)
  dst_p = _pad_edges(dst, _NPAD - 1)
  x_p = jnp.concatenate([x, jnp.zeros((_NPAD - _N, _D), jnp.float32)])

  y, ybf = _mm1(x_p, W1)
  part1, degp = _agg_h(ybf, src_p, dst_p)
  t, dinvb = _mid(y, part1, part1, degp, degp, b1.reshape(1, _H), W2)
  part2 = _agg_c(t, src_p, dst_p)
  logits = _fin(t, part2, part2, dinvb, b2.reshape(1, _C))
  return logits


# speedup vs baseline: 1.0879x; 1.0879x over previous
import functools

import jax
import jax.numpy as jnp
from jax import lax
from jax.experimental import pallas as pl
from jax.experimental.pallas import tpu as pltpu
from jax.experimental.pallas import tpu_sc as plsc

_N = 10000
_D = 128
_H = 64
_C = 16
_E = 320000

_NCORE = 2
_NSUB = 16
_SUB = 128
_NBJ = 80
_EPAD = _NCORE * _NSUB * _NBJ * _SUB
_NPAD = 10240
_RPS = _NPAD // _NSUB
_ZR = 128

_BMP = 2048
_BM = 2000

_SC_PARAMS = pltpu.CompilerParams(use_tc_tiling_on_sc=False)


def _make_agg(feat, with_deg, dt=jnp.float32):
  part_t = jax.ShapeDtypeStruct((_NCORE, _NPAD, feat), dt)
  lanes = 16 if dt == jnp.float32 else 32
  out_type = ([part_t, jax.ShapeDtypeStruct((_NCORE, 1, _NPAD), jnp.float32)]
              if with_deg else part_t)
  scratch = [
      pltpu.VMEM((_NBJ, _SUB), jnp.int32),
      pltpu.VMEM((_NBJ, _SUB), jnp.int32),
      pltpu.VMEM((_SUB, feat), dt),
      pltpu.VMEM((_SUB, feat), dt),
      pltpu.VMEM((_ZR, feat), dt),
      pltpu.SemaphoreType.DMA,
      pltpu.SemaphoreType.DMA,
      pltpu.VMEM_SHARED((_NPAD, feat), dt),
  ]
  if with_deg:
    scratch.append(pltpu.VMEM_SHARED((_NPAD,), jnp.float32))
    scratch.append(pltpu.VMEM((_SUB,), jnp.float32))
    scratch.append(pltpu.SemaphoreType.DMA)

  def body(y_hbm, src_hbm, dst_hbm, *rest):
    if with_deg:
      (part_out, deg_out, src_v, dst_v, rows0, rows1, zrow_v, sem0, sem1,
       acc_sh, deg_sh, ones_v, dsem) = rest
    else:
      (part_out, src_v, dst_v, rows0, rows1, zrow_v, sem0, sem1,
       acc_sh) = rest
    c = lax.axis_index("c")
    s = lax.axis_index("s")
    sl = pl.ds(s * _RPS, _RPS)

    def zfill(r, carry):
      for k in range(feat // lanes):
        zrow_v[r, pl.ds(k * lanes, lanes)] = jnp.zeros((lanes,), dt)
      return carry

    lax.fori_loop(0, _ZR, zfill, 0)
    for m in range(_RPS // _ZR):
      pltpu.sync_copy(zrow_v, acc_sh.at[pl.ds(s * _RPS + m * _ZR, _ZR)])
    if with_deg:
      for i in range(_SUB // 16):
        ones_v[pl.ds(i * 16, 16)] = jnp.zeros((16,), jnp.float32)
      for m in range(_RPS // _SUB):
        pltpu.sync_copy(ones_v, deg_sh.at[pl.ds(s * _RPS + m * _SUB, _SUB)])
      for i in range(_SUB // 16):
        ones_v[pl.ds(i * 16, 16)] = jnp.ones((16,), jnp.float32)
    pltpu.sync_copy(src_hbm.at[c, s], src_v)
    pltpu.sync_copy(dst_hbm.at[c, s], dst_v)
    plsc.subcore_barrier()

    pltpu.async_copy(y_hbm.at[src_v.at[0]], rows0, sem0)
    pltpu.async_copy(y_hbm.at[src_v.at[1]], rows1, sem1)

    def outer(jj, carry):
      for b, (rows, sem) in enumerate(((rows0, sem0), (rows1, sem1))):
        j = jj * 2 + b
        pltpu.make_async_copy(y_hbm.at[src_v.at[j]], rows, sem).wait()
        pltpu.sync_copy(rows, acc_sh.at[dst_v.at[j]], add=True)
        if with_deg:
          pltpu.async_copy(ones_v, deg_sh.at[dst_v.at[j]], dsem, add=True)

        @pl.when(j + 2 < _NBJ)
        def _():
          pltpu.async_copy(y_hbm.at[src_v.at[j + 2]], rows, sem)
      return carry

    lax.fori_loop(0, _NBJ // 2, outer, 0)
    if with_deg:
      def drain(j, carry):
        pltpu.make_async_copy(ones_v, deg_sh.at[dst_v.at[j]], dsem).wait()
        return carry
      lax.fori_loop(0, _NBJ, drain, 0)
    plsc.subcore_barrier()
    pltpu.sync_copy(acc_sh.at[sl], part_out.at[c, sl])
    if with_deg:
      pltpu.sync_copy(deg_sh.at[sl], deg_out.at[c, 0, sl])

  return functools.partial(
      pl.kernel, out_type=out_type,
      mesh=plsc.VectorSubcoreMesh(core_axis_name="c", subcore_axis_name="s"),
      scratch_types=scratch, compiler_params=_SC_PARAMS)(body)


_agg_h = _make_agg(_H, True, jnp.bfloat16)
_agg_c = _make_agg(_C, False)


def _mm1_body(x_ref, w_ref, o_ref, ob_ref):
  y = jnp.dot(x_ref[...], w_ref[...], preferred_element_type=jnp.float32)
  o_ref[...] = y
  ob_ref[...] = y.astype(jnp.bfloat16)


_mm1 = pl.pallas_call(
    _mm1_body,
    grid=(_NPAD // _BMP,),
    in_specs=[pl.BlockSpec((_BMP, _D), lambda i: (i, 0)),
              pl.BlockSpec((_D, _H), lambda i: (0, 0))],
    out_specs=[pl.BlockSpec((_BMP, _H), lambda i: (i, 0)),
               pl.BlockSpec((_BMP, _H), lambda i: (i, 0))],
    out_shape=[jax.ShapeDtypeStruct((_NPAD, _H), jnp.float32),
               jax.ShapeDtypeStruct((_NPAD, _H), jnp.bfloat16)],
)


def _mid_body(y_ref, p0_ref, p1_ref, dg0_ref, dg1_ref, b1_ref, w2_ref,
              t_ref, dinv_ref):
  deg = jnp.transpose(dg0_ref[0] + dg1_ref[0])
  dinv = jnp.where(deg > 0.0, 1.0 / jnp.maximum(deg, 1.0), 0.0)
  aggv = (p0_ref[0].astype(jnp.float32)
          + p1_ref[0].astype(jnp.float32)) * dinv
  h = jnp.maximum(0.5 * y_ref[...] + 0.5 * aggv + b1_ref[...], 0.0)
  t_ref[...] = jnp.dot(h, w2_ref[...], preferred_element_type=jnp.float32)
  dinv_ref[...] = jnp.broadcast_to(dinv, (_BMP, _C))


_mid = pl.pallas_call(
    _mid_body,
    grid=(_NPAD // _BMP,),
    in_specs=[pl.BlockSpec((_BMP, _H), lambda i: (i, 0)),
              pl.BlockSpec((1, _BMP, _H), lambda i: (0, i, 0)),
              pl.BlockSpec((1, _BMP, _H), lambda i: (1, i, 0)),
              pl.BlockSpec((1, 1, _BMP), lambda i: (0, 0, i)),
              pl.BlockSpec((1, 1, _BMP), lambda i: (1, 0, i)),
              pl.BlockSpec((1, _H), lambda i: (0, 0)),
              pl.BlockSpec((_H, _C), lambda i: (0, 0))],
    out_specs=[pl.BlockSpec((_BMP, _C), lambda i: (i, 0)),
               pl.BlockSpec((_BMP, _C), lambda i: (i, 0))],
    out_shape=[jax.ShapeDtypeStruct((_NPAD, _C), jnp.float32),
               jax.ShapeDtypeStruct((_NPAD, _C), jnp.float32)],
)


def _fin_body(t_ref, q0_ref, q1_ref, dinv_ref, b2_ref, o_ref):
  aggv = (q0_ref[0] + q1_ref[0]) * dinv_ref[...]
  o_ref[...] = 0.5 * t_ref[...] + 0.5 * aggv + b2_ref[...]


_fin = pl.pallas_call(
    _fin_body,
    grid=(_N // _BM,),
    in_specs=[pl.BlockSpec((_BM, _C), lambda i: (i, 0)),
              pl.BlockSpec((1, _BM, _C), lambda i: (0, i, 0)),
              pl.BlockSpec((1, _BM, _C), lambda i: (1, i, 0)),
              pl.BlockSpec((_BM, _C), lambda i: (i, 0)),
              pl.BlockSpec((1, _C), lambda i: (0, 0))],
    out_specs=pl.BlockSpec((_BM, _C), lambda i: (i, 0)),
    out_shape=jax.ShapeDtypeStruct((_N, _C), jnp.float32),
)


def _pad_edges(idx, fill):
  return jnp.concatenate(
      [idx, jnp.full((_EPAD - _E,), fill, jnp.int32)]).reshape(
          _NCORE, _NSUB, _NBJ, _SUB)


def kernel(x, edge_index, W1, b1, W2, b2):
  src = edge_index[0]
  dst = edge_index[1]
  src_p = _pad_edges(src, 0)
  dst_p = _pad_edges(dst, _NPAD - 1)
  x_p = jnp.concatenate([x, jnp.zeros((_NPAD - _N, _D), jnp.float32)])

  y, ybf = _mm1(x_p, W1)
  part1, degp = _agg_h(ybf, src_p, dst_p)
  t, dinvb = _mid(y, part1, part1, degp, degp, b1.reshape(1, _H), W2)
  part2 = _agg_c(t, src_p, dst_p)
  logits = _fin(t, part2, part2, dinvb, b2.reshape(1, _C))
  return logits
